# baseline (speedup 1.0000x reference)
import functools

import jax
import jax.numpy as jnp
from jax.experimental import pallas as pl
from jax.experimental.pallas import tpu as pltpu

N = 10000
E = 160000
NP = 10240
H = 512
OUT = 64
BR = 1024


def _dense_body(h_ref, agg_ref, wlt, bl, wrt, w1t, b1, w2t, b2, out_ref):
    t = (
        jnp.dot(agg_ref[...], wlt[...], preferred_element_type=jnp.float32)
        + bl[...]
        + jnp.dot(h_ref[...], wrt[...], preferred_element_type=jnp.float32)
    )
    h1 = jnp.maximum(
        jnp.dot(t, w1t[...], preferred_element_type=jnp.float32) + b1[...], 0.0
    )
    h2 = jnp.maximum(
        jnp.dot(h1, w2t[...], preferred_element_type=jnp.float32) + b2[...], 0.0
    )
    out_ref[...] = h2


def _final_body(h_ref, agg_ref, wlt, bl, wrt, w1t, b1, w2t, b2,
                fc1t, fc1b, fc2t, fc2b, out_ref):
    t = (
        jnp.dot(agg_ref[...], wlt[...], preferred_element_type=jnp.float32)
        + bl[...]
        + jnp.dot(h_ref[...], wrt[...], preferred_element_type=jnp.float32)
    )
    h1 = jnp.maximum(
        jnp.dot(t, w1t[...], preferred_element_type=jnp.float32) + b1[...], 0.0
    )
    h2 = jnp.maximum(
        jnp.dot(h1, w2t[...], preferred_element_type=jnp.float32) + b2[...], 0.0
    )
    f1 = jnp.maximum(
        jnp.dot(h2, fc1t[...], preferred_element_type=jnp.float32) + fc1b[...], 0.0
    )
    f2 = jnp.dot(f1, fc2t[...], preferred_element_type=jnp.float32) + fc2b[...]
    out_ref[...] = 1.0 / (1.0 + jnp.exp(-f2))


def _wspec(shape):
    return pl.BlockSpec(shape, lambda i: (0, 0))


def _dense_layer(h, agg, wlt, bl, wrt, w1t, b1, w2t, b2):
    din = h.shape[1]
    grid = (NP // BR,)
    return pl.pallas_call(
        _dense_body,
        grid=grid,
        in_specs=[
            pl.BlockSpec((BR, din), lambda i: (i, 0)),
            pl.BlockSpec((BR, din), lambda i: (i, 0)),
            _wspec((din, H)), _wspec((1, H)), _wspec((din, H)),
            _wspec((H, H)), _wspec((1, H)), _wspec((H, H)), _wspec((1, H)),
        ],
        out_specs=pl.BlockSpec((BR, H), lambda i: (i, 0)),
        out_shape=jax.ShapeDtypeStruct((NP, H), jnp.float32),
    )(h, agg, wlt, bl, wrt, w1t, b1, w2t, b2)


def _final_layer(h, agg, wlt, bl, wrt, w1t, b1, w2t, b2, fc1t, fc1b, fc2t, fc2b):
    din = h.shape[1]
    grid = (NP // BR,)
    return pl.pallas_call(
        _final_body,
        grid=grid,
        in_specs=[
            pl.BlockSpec((BR, din), lambda i: (i, 0)),
            pl.BlockSpec((BR, din), lambda i: (i, 0)),
            _wspec((din, H)), _wspec((1, H)), _wspec((din, H)),
            _wspec((H, H)), _wspec((1, H)), _wspec((H, H)), _wspec((1, H)),
            _wspec((H, H // 2)), _wspec((1, H // 2)),
            _wspec((H // 2, OUT)), _wspec((1, OUT)),
        ],
        out_specs=pl.BlockSpec((BR, OUT), lambda i: (i, 0)),
        out_shape=jax.ShapeDtypeStruct((NP, OUT), jnp.float32),
    )(h, agg, wlt, bl, wrt, w1t, b1, w2t, b2, fc1t, fc1b, fc2t, fc2b)


def _segment_sum(h, src, dst, din):
    msgs = jnp.take(h[:N], src, axis=0)
    agg = jax.ops.segment_sum(msgs, dst, num_segments=N)
    return jnp.pad(agg, ((0, NP - N), (0, 0)))


def kernel(x, edge_index,
           conv0_Wl, conv0_bl, conv0_Wr, mlp0_W1, mlp0_b1, mlp0_W2, mlp0_b2,
           conv1_Wl, conv1_bl, conv1_Wr, mlp1_W1, mlp1_b1, mlp1_W2, mlp1_b2,
           conv2_Wl, conv2_bl, conv2_Wr, mlp2_W1, mlp2_b1, mlp2_W2, mlp2_b2,
           fc1_W, fc1_b, fc2_W, fc2_b):
    src = edge_index[0]
    dst = edge_index[1]
    layers = [
        (conv0_Wl, conv0_bl, conv0_Wr, mlp0_W1, mlp0_b1, mlp0_W2, mlp0_b2),
        (conv1_Wl, conv1_bl, conv1_Wr, mlp1_W1, mlp1_b1, mlp1_W2, mlp1_b2),
        (conv2_Wl, conv2_bl, conv2_Wr, mlp2_W1, mlp2_b1, mlp2_W2, mlp2_b2),
    ]
    h = jnp.pad(x, ((0, NP - N), (0, 0)))
    for i in range(3):
        wl, bl, wr, w1, b1, w2, b2 = layers[i]
        args = (wl.T, bl[None, :], wr.T, w1.T, b1[None, :], w2.T, b2[None, :])
        agg = _segment_sum(h, src, dst, h.shape[1])
        if i < 2:
            h = _dense_layer(h, agg, *args)
        else:
            out = _final_layer(h, agg, *args,
                               fc1_W.T, fc1_b[None, :], fc2_W.T, fc2_b[None, :])
    return out[:N]


# speedup vs baseline: 4.0435x; 4.0435x over previous
import functools

import jax
import jax.numpy as jnp
from jax import lax
from jax.experimental import pallas as pl
from jax.experimental.pallas import tpu as pltpu
from jax.experimental.pallas import tpu_sc as plsc

N = 10000
E = 160000
NP = 10240
H = 512
OUT = 64
BR = 1024

NC = 2
NS = 16
L = 16
EPT = E // NS
RC = 2000
KSUB = 128
KEPT = EPT + 112
HALF = NP // NC
ACC_SUB = 10368
RPT = ACC_SUB // NS


def _make_sc_agg(D):
    S = D // 128
    KR = KSUB // S
    chunk_rows = 2560 if D == 512 else 5120
    qpc = HALF // chunk_rows
    real_sub = chunk_rows * S
    wpt = real_sub // NS

    mesh = plsc.VectorSubcoreMesh(core_axis_name="c", subcore_axis_name="s")

    @functools.partial(
        pl.kernel,
        out_type=jax.ShapeDtypeStruct((NP * S, 128), jnp.float32),
        mesh=mesh,
        scratch_types=[
            pltpu.VMEM((RC,), jnp.int32),
            pltpu.VMEM((RC,), jnp.int32),
            pltpu.VMEM((KEPT,), jnp.int32),
            pltpu.VMEM((KEPT,), jnp.int32),
            pltpu.VMEM((KSUB,), jnp.int32),
            pltpu.VMEM((KSUB,), jnp.int32),
            pltpu.VMEM((KSUB, 128), jnp.float32),
            pltpu.VMEM_SHARED((ACC_SUB, 128), jnp.float32),
            pltpu.SemaphoreType.DMA,
        ],
        compiler_params=pltpu.CompilerParams(needs_layout_passes=False),
    )
    def body(h_hbm, src_hbm, dst_hbm, zeros_hbm, out_hbm,
             raw_src, raw_dst, kept_src, kept_dst, idxg, idxd, gbuf,
             acc, sem):
        c = lax.axis_index("c")
        s = lax.axis_index("s")

        for qi in range(qpc):
            lo = (c * qpc + qi) * chunk_rows

            pltpu.sync_copy(zeros_hbm, acc.at[pl.ds(s * RPT, RPT)])
            plsc.subcore_barrier()

            def rchunk(ci, off):
                pltpu.sync_copy(src_hbm.at[pl.ds(s * EPT + ci * RC, RC)],
                                raw_src)
                pltpu.sync_copy(dst_hbm.at[pl.ds(s * EPT + ci * RC, RC)],
                                raw_dst)

                def fbody(i, off):
                    sv = raw_src[pl.ds(i * L, L)]
                    dv = raw_dst[pl.ds(i * L, L)]
                    m = (dv >= lo) & (dv < lo + chunk_rows)
                    mi = m.astype(jnp.int32)
                    pos = off + plsc.cumsum(mi) - 1
                    plsc.store_scatter(kept_src, [pos], sv, mask=m)
                    plsc.store_scatter(kept_dst, [pos], dv - lo, mask=m)
                    return off + jnp.sum(mi)
                return lax.fori_loop(0, RC // L, fbody, off)
            cnt = lax.fori_loop(0, EPT // RC, rchunk, 0)

            dummy = jnp.full((L,), chunk_rows, jnp.int32)
            zi = jnp.zeros((L,), jnp.int32)
            lane = lax.iota(jnp.int32, L)
            for t in range(KR // L):
                plsc.store_scatter(kept_src, [cnt + t * L + lane], zi)
                plsc.store_scatter(kept_dst, [cnt + t * L + lane], dummy)

            nb = (cnt + KR - 1) // KR

            def gbody(j, carry):
                o = j * KR
                for hh in range(KR // L):
                    sv = kept_src[pl.ds(o + hh * L, L)]
                    dv = kept_dst[pl.ds(o + hh * L, L)]
                    for t in range(S):
                        idxg[pl.ds(t * KR + hh * L, L)] = sv * S + t
                        idxd[pl.ds(t * KR + hh * L, L)] = dv * S + t
                pltpu.async_copy(h_hbm.at[idxg], gbuf, sem).wait()
                pltpu.sync_copy(gbuf, acc.at[idxd], add=True)
                return carry
            lax.fori_loop(0, nb, gbody, 0)

            plsc.subcore_barrier()
            pltpu.sync_copy(
                acc.at[pl.ds(s * wpt, wpt)],
                out_hbm.at[pl.ds((c * qpc + qi) * real_sub + s * wpt, wpt)])
            plsc.subcore_barrier()

    return body


_sc_agg_256 = _make_sc_agg(256)
_sc_agg_512 = _make_sc_agg(512)


def _dense_body(h_ref, agg_ref, wlt, bl, wrt, w1t, b1, w2t, b2, out_ref):
    t = (
        jnp.dot(agg_ref[...], wlt[...], preferred_element_type=jnp.float32)
        + bl[...]
        + jnp.dot(h_ref[...], wrt[...], preferred_element_type=jnp.float32)
    )
    h1 = jnp.maximum(
        jnp.dot(t, w1t[...], preferred_element_type=jnp.float32) + b1[...], 0.0
    )
    h2 = jnp.maximum(
        jnp.dot(h1, w2t[...], preferred_element_type=jnp.float32) + b2[...], 0.0
    )
    out_ref[...] = h2


def _final_body(h_ref, agg_ref, wlt, bl, wrt, w1t, b1, w2t, b2,
                fc1t, fc1b, fc2t, fc2b, out_ref):
    t = (
        jnp.dot(agg_ref[...], wlt[...], preferred_element_type=jnp.float32)
        + bl[...]
        + jnp.dot(h_ref[...], wrt[...], preferred_element_type=jnp.float32)
    )
    h1 = jnp.maximum(
        jnp.dot(t, w1t[...], preferred_element_type=jnp.float32) + b1[...], 0.0
    )
    h2 = jnp.maximum(
        jnp.dot(h1, w2t[...], preferred_element_type=jnp.float32) + b2[...], 0.0
    )
    f1 = jnp.maximum(
        jnp.dot(h2, fc1t[...], preferred_element_type=jnp.float32) + fc1b[...], 0.0
    )
    f2 = jnp.dot(f1, fc2t[...], preferred_element_type=jnp.float32) + fc2b[...]
    out_ref[...] = 1.0 / (1.0 + jnp.exp(-f2))


def _wspec(shape):
    return pl.BlockSpec(shape, lambda i: (0, 0))


def _dense_layer(h, agg, wlt, bl, wrt, w1t, b1, w2t, b2):
    din = h.shape[1]
    grid = (NP // BR,)
    return pl.pallas_call(
        _dense_body,
        grid=grid,
        in_specs=[
            pl.BlockSpec((BR, din), lambda i: (i, 0)),
            pl.BlockSpec((BR, din), lambda i: (i, 0)),
            _wspec((din, H)), _wspec((1, H)), _wspec((din, H)),
            _wspec((H, H)), _wspec((1, H)), _wspec((H, H)), _wspec((1, H)),
        ],
        out_specs=pl.BlockSpec((BR, H), lambda i: (i, 0)),
        out_shape=jax.ShapeDtypeStruct((NP, H), jnp.float32),
    )(h, agg, wlt, bl, wrt, w1t, b1, w2t, b2)


def _final_layer(h, agg, wlt, bl, wrt, w1t, b1, w2t, b2, fc1t, fc1b, fc2t, fc2b):
    din = h.shape[1]
    grid = (NP // BR,)
    return pl.pallas_call(
        _final_body,
        grid=grid,
        in_specs=[
            pl.BlockSpec((BR, din), lambda i: (i, 0)),
            pl.BlockSpec((BR, din), lambda i: (i, 0)),
            _wspec((din, H)), _wspec((1, H)), _wspec((din, H)),
            _wspec((H, H)), _wspec((1, H)), _wspec((H, H)), _wspec((1, H)),
            _wspec((H, H // 2)), _wspec((1, H // 2)),
            _wspec((H // 2, OUT)), _wspec((1, OUT)),
        ],
        out_specs=pl.BlockSpec((BR, OUT), lambda i: (i, 0)),
        out_shape=jax.ShapeDtypeStruct((NP, OUT), jnp.float32),
    )(h, agg, wlt, bl, wrt, w1t, b1, w2t, b2, fc1t, fc1b, fc2t, fc2b)


def _segment_sum(h, src, dst, din):
    fn = _sc_agg_256 if din == 256 else _sc_agg_512
    S = din // 128
    zeros = jnp.zeros((RPT, 128), jnp.float32)
    out = fn(h.reshape(NP * S, 128), src, dst, zeros)
    return out.reshape(NP, din)


def kernel(x, edge_index,
           conv0_Wl, conv0_bl, conv0_Wr, mlp0_W1, mlp0_b1, mlp0_W2, mlp0_b2,
           conv1_Wl, conv1_bl, conv1_Wr, mlp1_W1, mlp1_b1, mlp1_W2, mlp1_b2,
           conv2_Wl, conv2_bl, conv2_Wr, mlp2_W1, mlp2_b1, mlp2_W2, mlp2_b2,
           fc1_W, fc1_b, fc2_W, fc2_b):
    src = edge_index[0]
    dst = edge_index[1]
    layers = [
        (conv0_Wl, conv0_bl, conv0_Wr, mlp0_W1, mlp0_b1, mlp0_W2, mlp0_b2),
        (conv1_Wl, conv1_bl, conv1_Wr, mlp1_W1, mlp1_b1, mlp1_W2, mlp1_b2),
        (conv2_Wl, conv2_bl, conv2_Wr, mlp2_W1, mlp2_b1, mlp2_W2, mlp2_b2),
    ]
    h = jnp.pad(x, ((0, NP - N), (0, 0)))
    for i in range(3):
        wl, bl, wr, w1, b1, w2, b2 = layers[i]
        args = (wl.T, bl[None, :], wr.T, w1.T, b1[None, :], w2.T, b2[None, :])
        agg = _segment_sum(h, src, dst, h.shape[1])
        if i < 2:
            h = _dense_layer(h, agg, *args)
        else:
            out = _final_layer(h, agg, *args,
                               fc1_W.T, fc1_b[None, :], fc2_W.T, fc2_b[None, :])
    return out[:N]
